# baseline (speedup 1.0000x reference)
import functools

import jax
import jax.numpy as jnp
from jax import lax
from jax.experimental import pallas as pl
from jax.experimental.pallas import tpu as pltpu
from jax.experimental.pallas import tpu_sc as plsc

N = 100000
E = 1600000
D = 32
MLP_HID = 64
GRAPHS = 512
CATS = 128
BN_EPS = 1e-5

NP = 102400
NSC = NP // 1024
ZROWS = 3200
HALF = N // 2
ACC_ROWS = 50048
DUMP = HALF + 16
EPT = 98 * 1024
EP = 16 * EPT
CHUNK = 1024
KJ = CHUNK // 128
ECHUNK = 512
EKJ = ECHUNK // 128

POOL_ACC = 640
POOL_DUMP = GRAPHS + 32

RBLK = 5000
NBLK = N // RBLK

_mesh = plsc.VectorSubcoreMesh(core_axis_name="c", subcore_axis_name="s")
_sc_params = pltpu.CompilerParams(use_tc_tiling_on_sc=False)


@functools.partial(
    pl.kernel,
    out_type=jax.ShapeDtypeStruct((NP, D), jnp.float32),
    mesh=_mesh,
    scratch_types=[
        pltpu.VMEM((KJ, 128), jnp.int32),
        pltpu.VMEM((CHUNK, D), jnp.float32),
        pltpu.SemaphoreType.DMA,
    ],
    compiler_params=_sc_params,
)
def _emb_kernel(emb_hbm, x2d_hbm, h_hbm, idx, rows, sem):
    w = lax.axis_index("c") * 16 + lax.axis_index("s")
    for k in range(4):
        sc = k * 32 + w

        @pl.when(sc < NSC)
        def _():
            pltpu.sync_copy(x2d_hbm.at[pl.ds(sc * 8, KJ)], idx)
            descs = [
                pltpu.async_copy(emb_hbm.at[idx.at[j]],
                                 rows.at[pl.ds(j * 128, 128)], sem)
                for j in range(KJ)
            ]
            for d in descs:
                d.wait()
            pltpu.sync_copy(rows, h_hbm.at[pl.ds(sc * CHUNK, CHUNK)])


@functools.partial(
    pl.kernel,
    out_type=jax.ShapeDtypeStruct((N, D), jnp.float32),
    mesh=_mesh,
    scratch_types=[
        pltpu.VMEM((56, 128), jnp.int32),
        pltpu.VMEM((56, 128), jnp.int32),
        pltpu.VMEM((256, D), jnp.float32),
        pltpu.VMEM((256, D), jnp.float32),
        pltpu.VMEM_SHARED((ACC_ROWS, D), jnp.float32),
        pltpu.SemaphoreType.DMA,
        pltpu.SemaphoreType.DMA,
        pltpu.SemaphoreType.DMA,
    ],
    compiler_params=_sc_params,
)
def _agg_kernel(h_hbm, src2d_hbm, dst2d_hbm, zeros_hbm, agg_hbm,
                sidx, didx, rowsA, rowsB, acc, gsem, ssemA, ssemB):
    c = lax.axis_index("c")
    s = lax.axis_index("s")
    lo = c * HALF
    rows = (rowsA, rowsB)

    zrows = ACC_ROWS // 16
    pltpu.sync_copy(zeros_hbm.at[pl.ds(0, zrows)], acc.at[pl.ds(s * zrows, zrows)])
    plsc.subcore_barrier()

    tile_row = s * (EPT // 128)

    def remap(u):
        for r in (2 * u, 2 * u + 1):
            for i in range(8):
                v = didx[r, pl.ds(i * 16, 16)]
                ld = v - lo
                ok = (ld >= 0) & (ld < HALF)
                didx[r, pl.ds(i * 16, 16)] = jnp.where(ok, ld, DUMP)

    def fire_gather(u, buf):
        return [
            pltpu.async_copy(h_hbm.at[sidx.at[2 * u + r]],
                             buf.at[pl.ds(r * 128, 128)], gsem)
            for r in range(2)
        ]

    ssems = (ssemA, ssemB)

    def fire_scatter(u, buf):
        return [
            pltpu.async_copy(buf.at[pl.ds(r * 128, 128)],
                             acc.at[didx.at[2 * u + r]], ssems[u % 2],
                             add=True)
            for r in range(2)
        ]

    def body(b, carry):
        base = tile_row + b * 56
        pltpu.sync_copy(src2d_hbm.at[pl.ds(base, 56)], sidx)
        pltpu.sync_copy(dst2d_hbm.at[pl.ds(base, 56)], didx)
        remap(0)
        gd = fire_gather(0, rowsA)
        sd = None
        for u in range(28):
            buf = rows[u % 2]
            obuf = rows[1 - u % 2]
            for d in gd:
                d.wait()
            if u < 27:
                remap(u + 1)
            sd_new = fire_scatter(u, buf)
            if sd is not None:
                for d in sd:
                    d.wait()
            sd = sd_new
            if u < 27:
                gd = fire_gather(u + 1, obuf)
        for d in sd:
            d.wait()
        return carry

    lax.fori_loop(0, 14, body, 0)
    plsc.subcore_barrier()

    pltpu.sync_copy(acc.at[pl.ds(s * 3128, 3080)],
                    agg_hbm.at[pl.ds(lo + s * 3128, 3080)])

    @pl.when(s < 15)
    def _():
        pltpu.sync_copy(acc.at[pl.ds(s * 3128 + 3080, 48)],
                        agg_hbm.at[pl.ds(lo + s * 3128 + 3080, 48)])


@functools.partial(
    pl.kernel,
    out_type=jax.ShapeDtypeStruct((2, GRAPHS, D), jnp.float32),
    mesh=_mesh,
    scratch_types=[
        pltpu.VMEM((KJ, 128), jnp.int32),
        pltpu.VMEM((CHUNK, D), jnp.float32),
        pltpu.VMEM_SHARED((POOL_ACC, D), jnp.float32),
    ],
    compiler_params=_sc_params,
)
def _pool_kernel(h_hbm, b2d_hbm, zeros_hbm, out_hbm, idx, rows, acc):
    c = lax.axis_index("c")
    s = lax.axis_index("s")
    w = c * 16 + s

    zrows = POOL_ACC // 16
    pltpu.sync_copy(zeros_hbm.at[pl.ds(0, zrows)], acc.at[pl.ds(s * zrows, zrows)])
    plsc.subcore_barrier()

    for k in range(4):
        sc = k * 32 + w

        @pl.when(sc < NSC)
        def _():
            pltpu.sync_copy(b2d_hbm.at[pl.ds(sc * 8, KJ)], idx)
            pltpu.sync_copy(h_hbm.at[pl.ds(sc * CHUNK, CHUNK)], rows)
            for j in range(KJ):
                pltpu.sync_copy(rows.at[pl.ds(j * 128, 128)],
                                acc.at[idx.at[j]], add=True)
    plsc.subcore_barrier()

    orows = GRAPHS // 16
    pltpu.sync_copy(acc.at[pl.ds(s * orows, orows)],
                    out_hbm.at[c, pl.ds(s * orows, orows)])


def _k1_body(h_ref, agg_ref, w_ref, b_ref, o_ref):
    i = pl.program_id(0)
    z = h_ref[...] + agg_ref[...]
    o1 = jnp.dot(z, w_ref[...], preferred_element_type=jnp.float32) + b_ref[...]
    sv = jnp.sum(o1, axis=0)
    qv = jnp.sum(o1 * o1, axis=0)
    row = lax.broadcasted_iota(jnp.int32, (8, MLP_HID), 0)
    up = jnp.where(row == 0, sv[None, :], 0.0) + jnp.where(row == 1, qv[None, :], 0.0)

    @pl.when(i == 0)
    def _():
        o_ref[...] = up

    @pl.when(i != 0)
    def _():
        o_ref[...] = o_ref[...] + up


def _k2_body(h_ref, agg_ref, w1_ref, b1_ref, w2_ref, b2_ref, h2_ref, st_ref):
    i = pl.program_id(0)
    z = h_ref[...] + agg_ref[...]
    o1 = jnp.dot(z, w1_ref[...], preferred_element_type=jnp.float32) + b1_ref[...]
    t = jnp.maximum(o1, 0.0)
    h2 = jnp.dot(t, w2_ref[...], preferred_element_type=jnp.float32) + b2_ref[...]
    h2 = jnp.maximum(h2, 0.0)
    h2_ref[...] = h2
    sv = jnp.sum(h2, axis=0)
    qv = jnp.sum(h2 * h2, axis=0)
    row = lax.broadcasted_iota(jnp.int32, (8, D), 0)
    up = jnp.where(row == 0, sv[None, :], 0.0) + jnp.where(row == 1, qv[None, :], 0.0)

    @pl.when(i == 0)
    def _():
        st_ref[...] = up

    @pl.when(i != 0)
    def _():
        st_ref[...] = st_ref[...] + up


def _k3_body(h_ref, a_ref, c_ref, o_ref):
    o_ref[...] = h_ref[...] * a_ref[...] + c_ref[...]


def _head_body(p_ref, w1_ref, b1_ref, w2_ref, b2_ref, o_ref):
    g = p_ref[0] + p_ref[1]
    hd = jnp.dot(g, w1_ref[...], preferred_element_type=jnp.float32) + b1_ref[...]
    hd = jnp.maximum(hd, 0.0)
    o_ref[...] = jnp.dot(hd, w2_ref[...], preferred_element_type=jnp.float32) + b2_ref[...]


def _row_spec(width):
    return pl.BlockSpec((RBLK, width), lambda i: (i, 0))


def _full_spec(shape):
    return pl.BlockSpec(shape, lambda i: tuple(0 for _ in shape))


def _k1(h, agg, w1, b1):
    return pl.pallas_call(
        _k1_body,
        out_shape=jax.ShapeDtypeStruct((8, MLP_HID), jnp.float32),
        grid=(NBLK,),
        in_specs=[_row_spec(D), _row_spec(D), _full_spec((D, MLP_HID)),
                  _full_spec((1, MLP_HID))],
        out_specs=_full_spec((8, MLP_HID)),
    )(h, agg, w1, b1)


def _k2(h, agg, w1, b1, w2, b2):
    return pl.pallas_call(
        _k2_body,
        out_shape=(jax.ShapeDtypeStruct((NP, D), jnp.float32),
                   jax.ShapeDtypeStruct((8, D), jnp.float32)),
        grid=(NBLK,),
        in_specs=[_row_spec(D), _row_spec(D), _full_spec((D, MLP_HID)),
                  _full_spec((1, MLP_HID)), _full_spec((MLP_HID, D)),
                  _full_spec((1, D))],
        out_specs=(_row_spec(D), _full_spec((8, D))),
    )(h, agg, w1, b1, w2, b2)


def _k3(h2, a, c):
    return pl.pallas_call(
        _k3_body,
        out_shape=jax.ShapeDtypeStruct((NP, D), jnp.float32),
        grid=(NBLK,),
        in_specs=[_row_spec(D), _full_spec((1, D)), _full_spec((1, D))],
        out_specs=_row_spec(D),
    )(h2, a, c)


def _head(pool, w1, b1, w2, b2):
    return pl.pallas_call(
        _head_body,
        out_shape=jax.ShapeDtypeStruct((GRAPHS, 10), jnp.float32),
    )(pool, w1, b1, w2, b2)


def kernel(x, edge_index, batch, params):
    xp = jnp.pad(x, (0, NP - N)).reshape(NP // 128, 128)
    src = jnp.pad(edge_index[0], (0, EP - E)).reshape(EP // 128, 128)
    dst = jnp.pad(edge_index[1], (0, EP - E), constant_values=N).reshape(EP // 128, 128)
    bp = jnp.pad(batch, (0, NP - N), constant_values=GRAPHS).reshape(NP // 128, 128)
    zeros = jnp.zeros((ZROWS, D), jnp.float32)

    h = _emb_kernel(params['emb'], xp)
    for l in range(3):
        agg = _agg_kernel(h, src, dst, zeros)
        w1 = params[f'W1_{l}']
        b1 = params[f'b1_{l}'][None, :]
        st1 = _k1(h, agg, w1, b1)
        mean1 = st1[0] / N
        var1 = jnp.maximum(st1[1] / N - mean1 * mean1, 0.0)
        a1 = params[f'g1_{l}'] / jnp.sqrt(var1 + BN_EPS)
        c1 = params[f'be1_{l}'] - mean1 * a1
        w1s = w1 * a1[None, :]
        b1s = b1 * a1[None, :] + c1[None, :]
        h2, st2 = _k2(h, agg, w1s, b1s, params[f'W2_{l}'],
                      params[f'b2_{l}'][None, :])
        if l < 2:
            mean2 = st2[0] / N
            var2 = jnp.maximum(st2[1] / N - mean2 * mean2, 0.0)
            a2 = params[f'gbn_{l}'] / jnp.sqrt(var2 + BN_EPS)
            c2 = params[f'bbn_{l}'] - mean2 * a2
            h = _k3(h2, a2[None, :], c2[None, :])
        else:
            h = h2

    pool = _pool_kernel(h, bp, zeros)
    return _head(pool, params['Wh1'], params['bh1'][None, :],
                 params['Wh2'], params['bh2'][None, :])


# speedup vs baseline: 1.5716x; 1.5716x over previous
import functools

import jax
import jax.numpy as jnp
from jax import lax
from jax.experimental import pallas as pl
from jax.experimental.pallas import tpu as pltpu
from jax.experimental.pallas import tpu_sc as plsc

N = 100000
E = 1600000
D = 32
HD = 16
MLP_HID = 64
GRAPHS = 512
BN_EPS = 1e-5

NP = 102400
NSC = NP // 1024
ACC_ROWS = NP
EPT = 98 * 1024
EP = 16 * EPT
CHUNK = 1024
KJ = CHUNK // 128

POOL_ACC = 640

RBLK = 5000
NBLK = N // RBLK

_mesh = plsc.VectorSubcoreMesh(core_axis_name="c", subcore_axis_name="s")
_sc_params = pltpu.CompilerParams(use_tc_tiling_on_sc=False)


@functools.partial(
    pl.kernel,
    out_type=jax.ShapeDtypeStruct((2, NP, HD), jnp.float32),
    mesh=_mesh,
    scratch_types=[
        pltpu.VMEM((KJ, 128), jnp.int32),
        pltpu.VMEM((CHUNK, HD), jnp.float32),
        pltpu.VMEM((CHUNK, HD), jnp.float32),
        pltpu.SemaphoreType.DMA,
    ],
    compiler_params=_sc_params,
)
def _emb_kernel(emb_hbm, x2d_hbm, h_hbm, idx, rowsA, rowsB, sem):
    w = lax.axis_index("c") * 16 + lax.axis_index("s")
    for k in range(4):
        sc = k * 32 + w

        @pl.when(sc < NSC)
        def _():
            pltpu.sync_copy(x2d_hbm.at[pl.ds(sc * 8, KJ)], idx)
            descs = []
            for j in range(KJ):
                descs.append(pltpu.async_copy(
                    emb_hbm.at[0].at[idx.at[j]],
                    rowsA.at[pl.ds(j * 128, 128)], sem))
                descs.append(pltpu.async_copy(
                    emb_hbm.at[1].at[idx.at[j]],
                    rowsB.at[pl.ds(j * 128, 128)], sem))
            for d in descs:
                d.wait()
            pltpu.sync_copy(rowsA, h_hbm.at[0].at[pl.ds(sc * CHUNK, CHUNK)])
            pltpu.sync_copy(rowsB, h_hbm.at[1].at[pl.ds(sc * CHUNK, CHUNK)])


@functools.partial(
    pl.kernel,
    out_type=jax.ShapeDtypeStruct((2, N, HD), jnp.float32),
    mesh=_mesh,
    scratch_types=[
        pltpu.VMEM((56, 128), jnp.int32),
        pltpu.VMEM((56, 128), jnp.int32),
        pltpu.VMEM((256, HD), jnp.float32),
        pltpu.VMEM((256, HD), jnp.float32),
        pltpu.VMEM_SHARED((ACC_ROWS, HD), jnp.float32),
        pltpu.SemaphoreType.DMA,
        pltpu.SemaphoreType.DMA,
        pltpu.SemaphoreType.DMA,
    ],
    compiler_params=_sc_params,
)
def _agg_kernel(h_hbm, src2d_hbm, dst2d_hbm, zeros_hbm, agg_hbm,
                sidx, didx, rowsA, rowsB, acc, gsem, ssemA, ssemB):
    c = lax.axis_index("c")
    s = lax.axis_index("s")
    rows = (rowsA, rowsB)
    ssems = (ssemA, ssemB)
    hv = h_hbm.at[c]
    av = agg_hbm.at[c]

    zrows = ACC_ROWS // 16
    pltpu.sync_copy(zeros_hbm.at[pl.ds(0, zrows)], acc.at[pl.ds(s * zrows, zrows)])
    plsc.subcore_barrier()

    tile_row = s * (EPT // 128)

    def fire_gather(u, buf):
        return [
            pltpu.async_copy(hv.at[sidx.at[2 * u + r]],
                             buf.at[pl.ds(r * 128, 128)], gsem)
            for r in range(2)
        ]

    def fire_scatter(u, buf):
        return [
            pltpu.async_copy(buf.at[pl.ds(r * 128, 128)],
                             acc.at[didx.at[2 * u + r]], ssems[u % 2],
                             add=True)
            for r in range(2)
        ]

    def body(b, carry):
        base = tile_row + b * 56
        pltpu.sync_copy(src2d_hbm.at[pl.ds(base, 56)], sidx)
        pltpu.sync_copy(dst2d_hbm.at[pl.ds(base, 56)], didx)
        gd = fire_gather(0, rowsA)
        sd = None
        for u in range(28):
            buf = rows[u % 2]
            obuf = rows[1 - u % 2]
            for d in gd:
                d.wait()
            sd_new = fire_scatter(u, buf)
            if sd is not None:
                for d in sd:
                    d.wait()
            sd = sd_new
            if u < 27:
                gd = fire_gather(u + 1, obuf)
        for d in sd:
            d.wait()
        return carry

    lax.fori_loop(0, 14, body, 0)
    plsc.subcore_barrier()

    pltpu.sync_copy(acc.at[pl.ds(s * 6256, 6160)],
                    av.at[pl.ds(s * 6256, 6160)])

    @pl.when(s < 15)
    def _():
        pltpu.sync_copy(acc.at[pl.ds(s * 6256 + 6160, 96)],
                        av.at[pl.ds(s * 6256 + 6160, 96)])


@functools.partial(
    pl.kernel,
    out_type=jax.ShapeDtypeStruct((2, 2, GRAPHS, HD), jnp.float32),
    mesh=_mesh,
    scratch_types=[
        pltpu.VMEM((KJ, 128), jnp.int32),
        pltpu.VMEM((CHUNK, HD), jnp.float32),
        pltpu.VMEM((CHUNK, HD), jnp.float32),
        pltpu.VMEM_SHARED((POOL_ACC, HD), jnp.float32),
        pltpu.VMEM_SHARED((POOL_ACC, HD), jnp.float32),
    ],
    compiler_params=_sc_params,
)
def _pool_kernel(h_hbm, b2d_hbm, zeros_hbm, out_hbm,
                 idx, rowsA, rowsB, accA, accB):
    c = lax.axis_index("c")
    s = lax.axis_index("s")
    w = c * 16 + s

    zrows = POOL_ACC // 16
    pltpu.sync_copy(zeros_hbm.at[pl.ds(0, zrows)], accA.at[pl.ds(s * zrows, zrows)])
    pltpu.sync_copy(zeros_hbm.at[pl.ds(0, zrows)], accB.at[pl.ds(s * zrows, zrows)])
    plsc.subcore_barrier()

    for k in range(4):
        sc = k * 32 + w

        @pl.when(sc < NSC)
        def _():
            pltpu.sync_copy(b2d_hbm.at[pl.ds(sc * 8, KJ)], idx)
            pltpu.sync_copy(h_hbm.at[0].at[pl.ds(sc * CHUNK, CHUNK)], rowsA)
            pltpu.sync_copy(h_hbm.at[1].at[pl.ds(sc * CHUNK, CHUNK)], rowsB)
            for j in range(KJ):
                pltpu.sync_copy(rowsA.at[pl.ds(j * 128, 128)],
                                accA.at[idx.at[j]], add=True)
                pltpu.sync_copy(rowsB.at[pl.ds(j * 128, 128)],
                                accB.at[idx.at[j]], add=True)
    plsc.subcore_barrier()

    orows = GRAPHS // 16
    pltpu.sync_copy(accA.at[pl.ds(s * orows, orows)],
                    out_hbm.at[c].at[0].at[pl.ds(s * orows, orows)])
    pltpu.sync_copy(accB.at[pl.ds(s * orows, orows)],
                    out_hbm.at[c].at[1].at[pl.ds(s * orows, orows)])


def _zcat(h_ref, agg_ref):
    z = jnp.concatenate([h_ref[0], h_ref[1]], axis=1)
    a = jnp.concatenate([agg_ref[0], agg_ref[1]], axis=1)
    return z + a


def _k1_body(h_ref, agg_ref, w_ref, b_ref, o_ref):
    i = pl.program_id(0)
    z = _zcat(h_ref, agg_ref)
    o1 = jnp.dot(z, w_ref[...], preferred_element_type=jnp.float32) + b_ref[...]
    sv = jnp.sum(o1, axis=0)
    qv = jnp.sum(o1 * o1, axis=0)
    row = lax.broadcasted_iota(jnp.int32, (8, MLP_HID), 0)
    up = jnp.where(row == 0, sv[None, :], 0.0) + jnp.where(row == 1, qv[None, :], 0.0)

    @pl.when(i == 0)
    def _():
        o_ref[...] = up

    @pl.when(i != 0)
    def _():
        o_ref[...] = o_ref[...] + up


def _k2_body(h_ref, agg_ref, w1_ref, b1_ref, w2_ref, b2_ref, h2_ref, st_ref):
    i = pl.program_id(0)
    z = _zcat(h_ref, agg_ref)
    o1 = jnp.dot(z, w1_ref[...], preferred_element_type=jnp.float32) + b1_ref[...]
    t = jnp.maximum(o1, 0.0)
    h2 = jnp.dot(t, w2_ref[...], preferred_element_type=jnp.float32) + b2_ref[...]
    h2 = jnp.maximum(h2, 0.0)
    h2_ref[0] = h2[:, :HD]
    h2_ref[1] = h2[:, HD:]
    sv = jnp.sum(h2, axis=0)
    qv = jnp.sum(h2 * h2, axis=0)
    row = lax.broadcasted_iota(jnp.int32, (8, D), 0)
    up = jnp.where(row == 0, sv[None, :], 0.0) + jnp.where(row == 1, qv[None, :], 0.0)

    @pl.when(i == 0)
    def _():
        st_ref[...] = up

    @pl.when(i != 0)
    def _():
        st_ref[...] = st_ref[...] + up


def _k3_body(h_ref, a_ref, c_ref, o_ref):
    o_ref[0] = h_ref[0] * a_ref[:, :HD] + c_ref[:, :HD]
    o_ref[1] = h_ref[1] * a_ref[:, HD:] + c_ref[:, HD:]


def _head_body(p_ref, w1_ref, b1_ref, w2_ref, b2_ref, o_ref):
    g = jnp.concatenate([p_ref[0, 0] + p_ref[1, 0],
                         p_ref[0, 1] + p_ref[1, 1]], axis=1)
    hd = jnp.dot(g, w1_ref[...], preferred_element_type=jnp.float32) + b1_ref[...]
    hd = jnp.maximum(hd, 0.0)
    o_ref[...] = jnp.dot(hd, w2_ref[...], preferred_element_type=jnp.float32) + b2_ref[...]


def _split_spec():
    return pl.BlockSpec((2, RBLK, HD), lambda i: (0, i, 0))


def _full_spec(shape):
    return pl.BlockSpec(shape, lambda i: tuple(0 for _ in shape))


def _k1(h, agg, w1, b1):
    return pl.pallas_call(
        _k1_body,
        out_shape=jax.ShapeDtypeStruct((8, MLP_HID), jnp.float32),
        grid=(NBLK,),
        in_specs=[_split_spec(), _split_spec(), _full_spec((D, MLP_HID)),
                  _full_spec((1, MLP_HID))],
        out_specs=_full_spec((8, MLP_HID)),
    )(h, agg, w1, b1)


def _k2(h, agg, w1, b1, w2, b2):
    return pl.pallas_call(
        _k2_body,
        out_shape=(jax.ShapeDtypeStruct((2, NP, HD), jnp.float32),
                   jax.ShapeDtypeStruct((8, D), jnp.float32)),
        grid=(NBLK,),
        in_specs=[_split_spec(), _split_spec(), _full_spec((D, MLP_HID)),
                  _full_spec((1, MLP_HID)), _full_spec((MLP_HID, D)),
                  _full_spec((1, D))],
        out_specs=(_split_spec(), _full_spec((8, D))),
    )(h, agg, w1, b1, w2, b2)


def _k3(h2, a, c):
    return pl.pallas_call(
        _k3_body,
        out_shape=jax.ShapeDtypeStruct((2, NP, HD), jnp.float32),
        grid=(NBLK,),
        in_specs=[_split_spec(), _full_spec((1, D)), _full_spec((1, D))],
        out_specs=_split_spec(),
    )(h2, a, c)


def _head(pool, w1, b1, w2, b2):
    return pl.pallas_call(
        _head_body,
        out_shape=jax.ShapeDtypeStruct((GRAPHS, 10), jnp.float32),
    )(pool, w1, b1, w2, b2)


def kernel(x, edge_index, batch, params):
    xp = jnp.pad(x, (0, NP - N)).reshape(NP // 128, 128)
    src = jnp.pad(edge_index[0], (0, EP - E)).reshape(EP // 128, 128)
    dst = jnp.pad(edge_index[1], (0, EP - E), constant_values=N).reshape(EP // 128, 128)
    bp = jnp.pad(batch, (0, NP - N), constant_values=GRAPHS).reshape(NP // 128, 128)
    zeros = jnp.zeros((ACC_ROWS // 16, HD), jnp.float32)
    emb2 = params['emb'].reshape(128, 2, HD).transpose(1, 0, 2)

    h = _emb_kernel(emb2, xp)
    for l in range(3):
        agg = _agg_kernel(h, src, dst, zeros)
        w1 = params[f'W1_{l}']
        b1 = params[f'b1_{l}'][None, :]
        st1 = _k1(h, agg, w1, b1)
        mean1 = st1[0] / N
        var1 = jnp.maximum(st1[1] / N - mean1 * mean1, 0.0)
        a1 = params[f'g1_{l}'] / jnp.sqrt(var1 + BN_EPS)
        c1 = params[f'be1_{l}'] - mean1 * a1
        w1s = w1 * a1[None, :]
        b1s = b1 * a1[None, :] + c1[None, :]
        h2, st2 = _k2(h, agg, w1s, b1s, params[f'W2_{l}'],
                      params[f'b2_{l}'][None, :])
        if l < 2:
            mean2 = st2[0] / N
            var2 = jnp.maximum(st2[1] / N - mean2 * mean2, 0.0)
            a2 = params[f'gbn_{l}'] / jnp.sqrt(var2 + BN_EPS)
            c2 = params[f'bbn_{l}'] - mean2 * a2
            h = _k3(h2, a2[None, :], c2[None, :])
        else:
            h = h2

    pool = _pool_kernel(h, bp, zeros)
    return _head(pool, params['Wh1'], params['bh1'][None, :],
                 params['Wh2'], params['bh2'][None, :])
